# baseline (speedup 1.0000x reference)
import jax
import jax.numpy as jnp
import numpy as np
from jax import lax
from jax.experimental import pallas as pl
from jax.experimental.pallas import tpu as pltpu
from jax.experimental.pallas import tpu_sc as plsc

N = 10000
E = 320000
D = 128
ED = 16

NC = 2
NS = 16
NW = NC * NS
EH = 163840
EPT = EH // NW
C = 80
CH = EPT // C
RQ = 624


def _edge_proj_body(eat_ref, we_ref, be_ref, out_ref):
    out_ref[...] = (
        lax.dot_general(eat_ref[...], we_ref[...],
                        dimension_numbers=(((0,), (0,)), ((), ())),
                        preferred_element_type=jnp.float32)
        + be_ref[...]
    )


def _edge_proj(eaT, We, be, phase):
    BM = 1280
    HB = EH // BM
    return pl.pallas_call(
        _edge_proj_body,
        grid=(HB,),
        in_specs=[
            pl.BlockSpec((ED, BM), lambda i: (0, i + phase * HB)),
            pl.BlockSpec((ED, D), lambda i: (0, 0)),
            pl.BlockSpec((1, D), lambda i: (0, 0)),
        ],
        out_specs=pl.BlockSpec((BM, D), lambda i: (i, 0)),
        out_shape=jax.ShapeDtypeStruct((EH, D), jnp.float32),
    )(eaT, We, be.reshape(1, D))


def _sc_body(x_hbm, src_hbm, dst_hbm, e_hbm, out_hbm,
             src_vmem, d0, d1, x0, e0, x1, e1, accum,
             gs0, es0, ds0, ss0, gs1, es1, ds1, ss1):
    cid = lax.axis_index("c")
    sid = lax.axis_index("s")
    wid = sid * NC + cid
    ebase = wid * EPT

    def zrow(r, _):
        for k in range(D // 16):
            x0[r, pl.ds(k * 16, 16)] = jnp.zeros((16,), jnp.float32)
        return 0
    lax.fori_loop(0, C, zrow, 0)
    for j in range(RQ // C):
        pltpu.sync_copy(x0, accum.at[pl.ds(sid * RQ + j * C, C)])
    pltpu.sync_copy(x0.at[pl.ds(0, RQ - (RQ // C) * C)],
                    accum.at[pl.ds(sid * RQ + (RQ // C) * C,
                                   RQ - (RQ // C) * C)])
    for t in range(2):
        @pl.when(sid == t)
        def _():
            pltpu.sync_copy(x0.at[pl.ds(0, 8)],
                            accum.at[pl.ds(NS * RQ + t * 8, 8)])

    pltpu.sync_copy(src_hbm.at[pl.ds(ebase, EPT)], src_vmem)

    plsc.subcore_barrier()

    slots = ((x0, e0, d0, gs0, es0, ds0, ss0),
             (x1, e1, d1, gs1, es1, ds1, ss1))

    def start_data(c, s):
        xb, eb, db, gs, es, dsm, ss = s
        pltpu.async_copy(x_hbm.at[src_vmem.at[pl.ds(c * C, C)]], xb, gs)
        pltpu.async_copy(e_hbm.at[pl.ds(wid * EPT + c * C, C)], eb, es)

    def start_dst(c, s):
        xb, eb, db, gs, es, dsm, ss = s
        pltpu.async_copy(dst_hbm.at[pl.ds(ebase + c * C, C)], db, dsm)

    def wait_data(c, s):
        xb, eb, db, gs, es, dsm, ss = s
        pltpu.make_async_copy(
            x_hbm.at[src_vmem.at[pl.ds(c * C, C)]], xb, gs).wait()
        pltpu.make_async_copy(
            e_hbm.at[pl.ds(wid * EPT + c * C, C)], eb, es).wait()
        pltpu.make_async_copy(
            dst_hbm.at[pl.ds(ebase + c * C, C)], db, dsm).wait()

    def start_scatter(s):
        xb, eb, db, gs, es, dsm, ss = s
        pltpu.async_copy(xb, accum.at[db], ss, add=True)

    def wait_scatter(s):
        xb, eb, db, gs, es, dsm, ss = s
        pltpu.make_async_copy(xb, accum.at[db], ss).wait()

    def compute(s):
        xb, eb, db, gs, es, dsm, ss = s

        def rbody(r, _):
            zero16 = jnp.zeros((16,), jnp.float32)
            for k in range(D // 16):
                sl = pl.ds(k * 16, 16)
                xb[r, sl] = jnp.maximum(xb[r, sl] + eb[r, sl], zero16)
            return 0
        lax.fori_loop(0, C, rbody, 0)

    def process(c, cur, nxt):
        @pl.when(c >= 1)
        def _():
            wait_scatter(nxt)

        @pl.when(c + 1 < CH)
        def _():
            start_dst(c + 1, nxt)
            start_data(c + 1, nxt)

        wait_data(c, cur)
        compute(cur)
        start_scatter(cur)

    start_dst(0, slots[0])
    start_data(0, slots[0])

    def cbody(c, _):
        par = lax.rem(c, 2)

        @pl.when(par == 0)
        def _():
            process(c, slots[0], slots[1])

        @pl.when(par == 1)
        def _():
            process(c, slots[1], slots[0])

        return 0

    lax.fori_loop(0, CH, cbody, 0)
    wait_scatter(slots[(CH - 1) % 2])

    plsc.subcore_barrier()

    pltpu.sync_copy(accum.at[pl.ds(sid * RQ, RQ)],
                    out_hbm.at[cid, pl.ds(sid * RQ, RQ)])
    for t in range(2):
        @pl.when(sid == t)
        def _():
            r0 = NS * RQ + t * 8
            pltpu.sync_copy(accum.at[pl.ds(r0, 8)],
                            out_hbm.at[cid, pl.ds(r0, 8)])


def _sc_aggregate(x, srcg, dstg, e):
    mesh = plsc.VectorSubcoreMesh(core_axis_name="c", subcore_axis_name="s")
    kern = pl.kernel(
        _sc_body,
        out_type=jax.ShapeDtypeStruct((NC, N, D), jnp.float32),
        mesh=mesh,
        scratch_types=[
            pltpu.VMEM((EPT,), jnp.int32),
            pltpu.VMEM((C,), jnp.int32),
            pltpu.VMEM((C,), jnp.int32),
            pltpu.VMEM((C, D), jnp.float32),
            pltpu.VMEM((C, D), jnp.float32),
            pltpu.VMEM((C, D), jnp.float32),
            pltpu.VMEM((C, D), jnp.float32),
            pltpu.VMEM_SHARED((N + 16, D), jnp.float32),
            pltpu.SemaphoreType.DMA,
            pltpu.SemaphoreType.DMA,
            pltpu.SemaphoreType.DMA,
            pltpu.SemaphoreType.DMA,
            pltpu.SemaphoreType.DMA,
            pltpu.SemaphoreType.DMA,
            pltpu.SemaphoreType.DMA,
            pltpu.SemaphoreType.DMA,
        ],
    )
    return kern(x, srcg, dstg, e)


def _tail_body(x_ref, a0_ref, a1_ref, b0_ref, b1_ref, imp_ref, wn_ref, bn_ref,
               wgc_ref, wgi_ref, bg_ref, wp_ref, bp_ref,
               out_ref, prop_ref):
    x = x_ref[...]
    h = x + (a0_ref[0] + a1_ref[0]) + (b0_ref[0] + b1_ref[0])
    conv = jnp.dot(h, wn_ref[...], preferred_element_type=jnp.float32) + bn_ref[...]
    z = (jnp.dot(conv, wgc_ref[...], preferred_element_type=jnp.float32)
         + imp_ref[...] * wgi_ref[...] + bg_ref[...])
    gate = jax.nn.sigmoid(z)
    o = gate * conv + (1.0 - gate) * x
    out_ref[...] = o
    prop_ref[...] = jnp.sum(o * wp_ref[...], axis=1, keepdims=True) + bp_ref[...]


def _tail(x, pa, pb, importance, Wn, bn, Wg, bg, Wp, bp):
    BM = 2000
    grid = (N // BM,)
    return pl.pallas_call(
        _tail_body,
        grid=grid,
        in_specs=[
            pl.BlockSpec((BM, D), lambda i: (i, 0)),
            pl.BlockSpec((1, BM, D), lambda i: (0, i, 0)),
            pl.BlockSpec((1, BM, D), lambda i: (1, i, 0)),
            pl.BlockSpec((1, BM, D), lambda i: (0, i, 0)),
            pl.BlockSpec((1, BM, D), lambda i: (1, i, 0)),
            pl.BlockSpec((BM, 1), lambda i: (i, 0)),
            pl.BlockSpec((D, D), lambda i: (0, 0)),
            pl.BlockSpec((1, D), lambda i: (0, 0)),
            pl.BlockSpec((D, D), lambda i: (0, 0)),
            pl.BlockSpec((1, D), lambda i: (0, 0)),
            pl.BlockSpec((1, D), lambda i: (0, 0)),
            pl.BlockSpec((1, D), lambda i: (0, 0)),
            pl.BlockSpec((1, 1), lambda i: (0, 0)),
        ],
        out_specs=[
            pl.BlockSpec((BM, D), lambda i: (i, 0)),
            pl.BlockSpec((BM, 1), lambda i: (i, 0)),
        ],
        out_shape=[
            jax.ShapeDtypeStruct((N, D), jnp.float32),
            jax.ShapeDtypeStruct((N, 1), jnp.float32),
        ],
    )(x, pa, pa, pb, pb, importance, Wn, bn.reshape(1, D),
      Wg[:D], Wg[D:D + 1], bg.reshape(1, D), Wp.reshape(1, D),
      bp.reshape(1, 1))


_PERM = np.empty((D,), dtype=np.int32)
for _g in range(D // 32):
    for _i in range(16):
        _PERM[32 * _g + 2 * _i] = 32 * _g + _i
        _PERM[32 * _g + 2 * _i + 1] = 32 * _g + 16 + _i


def kernel(x, edge_index, edge_attr, importance, We, be, Wn, bn, Wg, bg, Wp, bp):
    eaT = edge_attr.T
    src, dst = edge_index[0], edge_index[1]
    half = E // 2
    pad_s = jnp.zeros((EH - half,), jnp.int32)
    pad_d = N + (jnp.arange(EH - half, dtype=jnp.int32) % 16)
    src_a = jnp.concatenate([src[:half], pad_s])
    dst_a = jnp.concatenate([dst[:half], pad_d])
    src_b = jnp.concatenate([src[half:], pad_s])
    dst_b = jnp.concatenate([dst[half:], pad_d])
    zc = jnp.zeros((ED, EH - half), jnp.float32)
    ea2 = jnp.concatenate([eaT[:, :half], zc, eaT[:, half:], zc], axis=1)
    e_a = _edge_proj(ea2, We, be, 0)
    p_a = _sc_aggregate(x, src_a, dst_a, e_a)
    e_b = _edge_proj(ea2, We, be, 1)
    p_b = _sc_aggregate(x, src_b, dst_b, e_b)
    out, prop = _tail(x, p_a, p_b, importance, Wn, bn, Wg, bg, Wp, bp)
    return (out, prop)


# speedup vs baseline: 2.0902x; 2.0902x over previous
import jax
import jax.numpy as jnp
import numpy as np
from jax import lax
from jax.experimental import pallas as pl
from jax.experimental.pallas import tpu as pltpu
from jax.experimental.pallas import tpu_sc as plsc

N = 10000
E = 320000
D = 128
ED = 16

NC = 2
NS = 16
NW = NC * NS
EH = 163840
EPT = EH // NW
C = 80
CH = EPT // C
RQ = 624


def _edge_proj_body(eat_ref, we_ref, be_ref, out_ref):
    out_ref[...] = (
        lax.dot_general(eat_ref[...], we_ref[...],
                        dimension_numbers=(((0,), (0,)), ((), ())),
                        preferred_element_type=jnp.float32)
        + be_ref[...]
    )


def _edge_proj(eaT, We, be, phase):
    BM = 1280
    HB = (E // 2) // BM
    return pl.pallas_call(
        _edge_proj_body,
        grid=(HB,),
        in_specs=[
            pl.BlockSpec((ED, BM), lambda i: (0, i + phase * HB)),
            pl.BlockSpec((ED, D), lambda i: (0, 0)),
            pl.BlockSpec((1, D), lambda i: (0, 0)),
        ],
        out_specs=pl.BlockSpec((BM, D), lambda i: (i, 0)),
        out_shape=jax.ShapeDtypeStruct((EH, D), jnp.float32),
    )(eaT, We, be.reshape(1, D))


def _sc_body(x_hbm, src_hbm, dst_hbm, e_hbm, out_hbm,
             src_vmem, d0, d1, x0, e0, x1, e1, accum,
             gs0, es0, ds0, ss0, gs1, es1, ds1, ss1):
    cid = lax.axis_index("c")
    sid = lax.axis_index("s")
    wid = sid * NC + cid
    ebase = wid * EPT

    def zrow(r, _):
        for k in range(D // 16):
            x0[r, pl.ds(k * 16, 16)] = jnp.zeros((16,), jnp.float32)
        return 0
    lax.fori_loop(0, C, zrow, 0)
    for j in range(RQ // C):
        pltpu.sync_copy(x0, accum.at[pl.ds(sid * RQ + j * C, C)])
    pltpu.sync_copy(x0.at[pl.ds(0, RQ - (RQ // C) * C)],
                    accum.at[pl.ds(sid * RQ + (RQ // C) * C,
                                   RQ - (RQ // C) * C)])
    for t in range(2):
        @pl.when(sid == t)
        def _():
            pltpu.sync_copy(x0.at[pl.ds(0, 8)],
                            accum.at[pl.ds(NS * RQ + t * 8, 8)])

    pltpu.sync_copy(src_hbm.at[pl.ds(ebase, EPT)], src_vmem)

    plsc.subcore_barrier()

    slots = ((x0, e0, d0, gs0, es0, ds0, ss0),
             (x1, e1, d1, gs1, es1, ds1, ss1))

    def start_data(c, s):
        xb, eb, db, gs, es, dsm, ss = s
        pltpu.async_copy(x_hbm.at[src_vmem.at[pl.ds(c * C, C)]], xb, gs)
        pltpu.async_copy(e_hbm.at[pl.ds(wid * EPT + c * C, C)], eb, es)

    def start_dst(c, s):
        xb, eb, db, gs, es, dsm, ss = s
        pltpu.async_copy(dst_hbm.at[pl.ds(ebase + c * C, C)], db, dsm)

    def wait_data(c, s):
        xb, eb, db, gs, es, dsm, ss = s
        pltpu.make_async_copy(
            x_hbm.at[src_vmem.at[pl.ds(c * C, C)]], xb, gs).wait()
        pltpu.make_async_copy(
            e_hbm.at[pl.ds(wid * EPT + c * C, C)], eb, es).wait()
        pltpu.make_async_copy(
            dst_hbm.at[pl.ds(ebase + c * C, C)], db, dsm).wait()

    def start_scatter(s):
        xb, eb, db, gs, es, dsm, ss = s
        pltpu.async_copy(xb, accum.at[db], ss, add=True)

    def wait_scatter(s):
        xb, eb, db, gs, es, dsm, ss = s
        pltpu.make_async_copy(xb, accum.at[db], ss).wait()

    def compute(s):
        xb, eb, db, gs, es, dsm, ss = s

        def rbody(r, _):
            zero16 = jnp.zeros((16,), jnp.float32)
            for k in range(D // 16):
                sl = pl.ds(k * 16, 16)
                xb[r, sl] = jnp.maximum(xb[r, sl] + eb[r, sl], zero16)
            return 0
        lax.fori_loop(0, C, rbody, 0)

    def process(c, cur, nxt, nch):
        @pl.when(c < nch)
        def _():
            @pl.when(c >= 1)
            def _():
                wait_scatter(nxt)

            @pl.when(c + 1 < nch)
            def _():
                start_dst(c + 1, nxt)
                start_data(c + 1, nxt)

            wait_data(c, cur)
            compute(cur)
            start_scatter(cur)

    rch = ((E // 2) - (NW - 1) * EPT + C - 1) // C
    nch = jnp.where(wid == NW - 1, rch, CH)

    start_dst(0, slots[0])
    start_data(0, slots[0])

    def cbody(c, _):
        par = lax.rem(c, 2)

        @pl.when(par == 0)
        def _():
            process(c, slots[0], slots[1], nch)

        @pl.when(par == 1)
        def _():
            process(c, slots[1], slots[0], nch)

        return 0

    lax.fori_loop(0, CH, cbody, 0)
    wait_scatter(slots[(CH - 1) % 2])

    plsc.subcore_barrier()

    pltpu.sync_copy(accum.at[pl.ds(sid * RQ, RQ)],
                    out_hbm.at[cid, pl.ds(sid * RQ, RQ)])
    for t in range(2):
        @pl.when(sid == t)
        def _():
            r0 = NS * RQ + t * 8
            pltpu.sync_copy(accum.at[pl.ds(r0, 8)],
                            out_hbm.at[cid, pl.ds(r0, 8)])


def _sc_aggregate(x, srcg, dstg, e):
    mesh = plsc.VectorSubcoreMesh(core_axis_name="c", subcore_axis_name="s")
    kern = pl.kernel(
        _sc_body,
        out_type=jax.ShapeDtypeStruct((NC, N, D), jnp.float32),
        mesh=mesh,
        scratch_types=[
            pltpu.VMEM((EPT,), jnp.int32),
            pltpu.VMEM((C,), jnp.int32),
            pltpu.VMEM((C,), jnp.int32),
            pltpu.VMEM((C, D), jnp.float32),
            pltpu.VMEM((C, D), jnp.float32),
            pltpu.VMEM((C, D), jnp.float32),
            pltpu.VMEM((C, D), jnp.float32),
            pltpu.VMEM_SHARED((N + 16, D), jnp.float32),
            pltpu.SemaphoreType.DMA,
            pltpu.SemaphoreType.DMA,
            pltpu.SemaphoreType.DMA,
            pltpu.SemaphoreType.DMA,
            pltpu.SemaphoreType.DMA,
            pltpu.SemaphoreType.DMA,
            pltpu.SemaphoreType.DMA,
            pltpu.SemaphoreType.DMA,
        ],
    )
    return kern(x, srcg, dstg, e)


def _tail_body(x_ref, a0_ref, a1_ref, b0_ref, b1_ref, imp_ref, wn_ref, bn_ref,
               wgc_ref, wgi_ref, bg_ref, wp_ref, bp_ref,
               out_ref, prop_ref):
    x = x_ref[...]
    h = x + (a0_ref[0] + a1_ref[0]) + (b0_ref[0] + b1_ref[0])
    conv = jnp.dot(h, wn_ref[...], preferred_element_type=jnp.float32) + bn_ref[...]
    z = (jnp.dot(conv, wgc_ref[...], preferred_element_type=jnp.float32)
         + imp_ref[...] * wgi_ref[...] + bg_ref[...])
    gate = jax.nn.sigmoid(z)
    o = gate * conv + (1.0 - gate) * x
    out_ref[...] = o
    prop_ref[...] = jnp.sum(o * wp_ref[...], axis=1, keepdims=True) + bp_ref[...]


def _tail(x, pa, pb, importance, Wn, bn, Wg, bg, Wp, bp):
    BM = 2000
    grid = (N // BM,)
    return pl.pallas_call(
        _tail_body,
        grid=grid,
        in_specs=[
            pl.BlockSpec((BM, D), lambda i: (i, 0)),
            pl.BlockSpec((1, BM, D), lambda i: (0, i, 0)),
            pl.BlockSpec((1, BM, D), lambda i: (1, i, 0)),
            pl.BlockSpec((1, BM, D), lambda i: (0, i, 0)),
            pl.BlockSpec((1, BM, D), lambda i: (1, i, 0)),
            pl.BlockSpec((BM, 1), lambda i: (i, 0)),
            pl.BlockSpec((D, D), lambda i: (0, 0)),
            pl.BlockSpec((1, D), lambda i: (0, 0)),
            pl.BlockSpec((D, D), lambda i: (0, 0)),
            pl.BlockSpec((1, D), lambda i: (0, 0)),
            pl.BlockSpec((1, D), lambda i: (0, 0)),
            pl.BlockSpec((1, D), lambda i: (0, 0)),
            pl.BlockSpec((1, 1), lambda i: (0, 0)),
        ],
        out_specs=[
            pl.BlockSpec((BM, D), lambda i: (i, 0)),
            pl.BlockSpec((BM, 1), lambda i: (i, 0)),
        ],
        out_shape=[
            jax.ShapeDtypeStruct((N, D), jnp.float32),
            jax.ShapeDtypeStruct((N, 1), jnp.float32),
        ],
    )(x, pa, pa, pb, pb, importance, Wn, bn.reshape(1, D),
      Wg[:D], Wg[D:D + 1], bg.reshape(1, D), Wp.reshape(1, D),
      bp.reshape(1, 1))


_PERM = np.empty((D,), dtype=np.int32)
for _g in range(D // 32):
    for _i in range(16):
        _PERM[32 * _g + 2 * _i] = 32 * _g + _i
        _PERM[32 * _g + 2 * _i + 1] = 32 * _g + 16 + _i


def kernel(x, edge_index, edge_attr, importance, We, be, Wn, bn, Wg, bg, Wp, bp):
    eaT = edge_attr.T
    src, dst = edge_index[0], edge_index[1]
    half = E // 2
    pad_s = jnp.zeros((EH - half,), jnp.int32)
    pad_d = N + (jnp.arange(EH - half, dtype=jnp.int32) % 16)
    src_a = jnp.concatenate([src[:half], pad_s])
    dst_a = jnp.concatenate([dst[:half], pad_d])
    src_b = jnp.concatenate([src[half:], pad_s])
    dst_b = jnp.concatenate([dst[half:], pad_d])
    e_a = _edge_proj(eaT, We, be, 0)
    p_a = _sc_aggregate(x, src_a, dst_a, e_a)
    e_b = _edge_proj(eaT, We, be, 1)
    p_b = _sc_aggregate(x, src_b, dst_b, e_b)
    out, prop = _tail(x, p_a, p_b, importance, Wn, bn, Wg, bg, Wp, bp)
    return (out, prop)


# speedup vs baseline: 2.1313x; 1.0197x over previous
import jax
import jax.numpy as jnp
import numpy as np
from jax import lax
from jax.experimental import pallas as pl
from jax.experimental.pallas import tpu as pltpu
from jax.experimental.pallas import tpu_sc as plsc

N = 10000
E = 320000
D = 128
ED = 16

NC = 2
NS = 16
NW = NC * NS
EPT = 5120
C = 80
CH = EPT // C
RQ = 624


def _edge_proj_body(eat_ref, we_ref, be_ref, out_ref):
    out_ref[...] = (
        lax.dot_general(eat_ref[...], we_ref[...],
                        dimension_numbers=(((0,), (0,)), ((), ())),
                        preferred_element_type=jnp.float32)
        + be_ref[...]
    )


def _edge_proj(eaT, We, be, phase):
    BM = 1280
    HB = (E // 2) // BM
    return pl.pallas_call(
        _edge_proj_body,
        grid=(HB,),
        in_specs=[
            pl.BlockSpec((ED, BM), lambda i: (0, i + phase * HB)),
            pl.BlockSpec((ED, D), lambda i: (0, 0)),
            pl.BlockSpec((1, D), lambda i: (0, 0)),
        ],
        out_specs=pl.BlockSpec((BM, D), lambda i: (i, 0)),
        out_shape=jax.ShapeDtypeStruct((E // 2, D), jnp.float32),
    )(eaT, We, be.reshape(1, D))


def _sc_body(phase, x_hbm, src_hbm, dst_hbm, e_hbm, out_hbm,
             src_vmem, d0, d1, x0, e0, x1, e1, accum,
             gs0, es0, ds0, ss0, gs1, es1, ds1, ss1):
    cid = lax.axis_index("c")
    sid = lax.axis_index("s")
    wid = sid * NC + cid
    base_r = phase * (E // 2) + wid * EPT
    sbase = pl.multiple_of(jnp.minimum(base_r, E - EPT), 128)
    delta = base_r - sbase

    def zrow(r, _):
        for k in range(D // 16):
            x0[r, pl.ds(k * 16, 16)] = jnp.zeros((16,), jnp.float32)
        return 0
    lax.fori_loop(0, C, zrow, 0)
    for j in range(RQ // C):
        pltpu.sync_copy(x0, accum.at[pl.ds(sid * RQ + j * C, C)])
    pltpu.sync_copy(x0.at[pl.ds(0, RQ - (RQ // C) * C)],
                    accum.at[pl.ds(sid * RQ + (RQ // C) * C,
                                   RQ - (RQ // C) * C)])
    for t in range(2):
        @pl.when(sid == t)
        def _():
            pltpu.sync_copy(x0.at[pl.ds(0, 8)],
                            accum.at[pl.ds(NS * RQ + t * 8, 8)])

    pltpu.sync_copy(src_hbm.at[pl.ds(sbase, EPT)], src_vmem)

    plsc.subcore_barrier()

    slots = ((x0, e0, d0, gs0, es0, ds0, ss0),
             (x1, e1, d1, gs1, es1, ds1, ss1))

    def start_data(c, s):
        xb, eb, db, gs, es, dsm, ss = s
        off = pl.multiple_of(delta + c * C, 8)
        pltpu.async_copy(x_hbm.at[src_vmem.at[pl.ds(off, C)]], xb, gs)
        pltpu.async_copy(e_hbm.at[pl.ds(wid * EPT + c * C, C)], eb, es)

    def start_dst(c, s):
        xb, eb, db, gs, es, dsm, ss = s
        pltpu.async_copy(dst_hbm.at[pl.ds(base_r + c * C, C)], db, dsm)

    def wait_data(c, s):
        xb, eb, db, gs, es, dsm, ss = s
        off = pl.multiple_of(delta + c * C, 8)
        pltpu.make_async_copy(
            x_hbm.at[src_vmem.at[pl.ds(off, C)]], xb, gs).wait()
        pltpu.make_async_copy(
            e_hbm.at[pl.ds(wid * EPT + c * C, C)], eb, es).wait()
        pltpu.make_async_copy(
            dst_hbm.at[pl.ds(base_r + c * C, C)], db, dsm).wait()

    def start_scatter(s):
        xb, eb, db, gs, es, dsm, ss = s
        pltpu.async_copy(xb, accum.at[db], ss, add=True)

    def wait_scatter(s):
        xb, eb, db, gs, es, dsm, ss = s
        pltpu.make_async_copy(xb, accum.at[db], ss).wait()

    def compute(s):
        xb, eb, db, gs, es, dsm, ss = s

        def rbody(r, _):
            zero16 = jnp.zeros((16,), jnp.float32)
            for k in range(D // 16):
                sl = pl.ds(k * 16, 16)
                xb[r, sl] = jnp.maximum(xb[r, sl] + eb[r, sl], zero16)
            return 0
        lax.fori_loop(0, C, rbody, 0)

    def process(c, cur, nxt, nch):
        @pl.when(c < nch)
        def _():
            @pl.when(c >= 1)
            def _():
                wait_scatter(nxt)

            @pl.when(c + 1 < nch)
            def _():
                start_dst(c + 1, nxt)
                start_data(c + 1, nxt)

            wait_data(c, cur)
            compute(cur)
            start_scatter(cur)

    rch = ((E // 2) - (NW - 1) * EPT + C - 1) // C
    nch = jnp.where(wid == NW - 1, rch, CH)

    start_dst(0, slots[0])
    start_data(0, slots[0])

    def cbody(c, _):
        par = lax.rem(c, 2)

        @pl.when(par == 0)
        def _():
            process(c, slots[0], slots[1], nch)

        @pl.when(par == 1)
        def _():
            process(c, slots[1], slots[0], nch)

        return 0

    lax.fori_loop(0, CH, cbody, 0)
    wait_scatter(slots[(CH - 1) % 2])

    plsc.subcore_barrier()

    pltpu.sync_copy(accum.at[pl.ds(sid * RQ, RQ)],
                    out_hbm.at[cid, pl.ds(sid * RQ, RQ)])
    for t in range(2):
        @pl.when(sid == t)
        def _():
            r0 = NS * RQ + t * 8
            pltpu.sync_copy(accum.at[pl.ds(r0, 8)],
                            out_hbm.at[cid, pl.ds(r0, 8)])


def _sc_aggregate(x, srcg, dstg, e, phase):
    import functools
    mesh = plsc.VectorSubcoreMesh(core_axis_name="c", subcore_axis_name="s")
    kern = pl.kernel(
        functools.partial(_sc_body, phase),
        out_type=jax.ShapeDtypeStruct((NC, N, D), jnp.float32),
        mesh=mesh,
        scratch_types=[
            pltpu.VMEM((EPT,), jnp.int32),
            pltpu.VMEM((C,), jnp.int32),
            pltpu.VMEM((C,), jnp.int32),
            pltpu.VMEM((C, D), jnp.float32),
            pltpu.VMEM((C, D), jnp.float32),
            pltpu.VMEM((C, D), jnp.float32),
            pltpu.VMEM((C, D), jnp.float32),
            pltpu.VMEM_SHARED((N, D), jnp.float32),
            pltpu.SemaphoreType.DMA,
            pltpu.SemaphoreType.DMA,
            pltpu.SemaphoreType.DMA,
            pltpu.SemaphoreType.DMA,
            pltpu.SemaphoreType.DMA,
            pltpu.SemaphoreType.DMA,
            pltpu.SemaphoreType.DMA,
            pltpu.SemaphoreType.DMA,
        ],
    )
    return kern(x, srcg, dstg, e)


def _tail_body(x_ref, a0_ref, a1_ref, b0_ref, b1_ref, imp_ref, wn_ref, bn_ref,
               wgc_ref, wgi_ref, bg_ref, wp_ref, bp_ref,
               out_ref, prop_ref):
    x = x_ref[...]
    h = x + (a0_ref[0] + a1_ref[0]) + (b0_ref[0] + b1_ref[0])
    conv = jnp.dot(h, wn_ref[...], preferred_element_type=jnp.float32) + bn_ref[...]
    z = (jnp.dot(conv, wgc_ref[...], preferred_element_type=jnp.float32)
         + imp_ref[...] * wgi_ref[...] + bg_ref[...])
    gate = jax.nn.sigmoid(z)
    o = gate * conv + (1.0 - gate) * x
    out_ref[...] = o
    prop_ref[...] = jnp.sum(o * wp_ref[...], axis=1, keepdims=True) + bp_ref[...]


def _tail(x, pa, pb, importance, Wn, bn, Wg, bg, Wp, bp):
    BM = 2000
    grid = (N // BM,)
    return pl.pallas_call(
        _tail_body,
        grid=grid,
        in_specs=[
            pl.BlockSpec((BM, D), lambda i: (i, 0)),
            pl.BlockSpec((1, BM, D), lambda i: (0, i, 0)),
            pl.BlockSpec((1, BM, D), lambda i: (1, i, 0)),
            pl.BlockSpec((1, BM, D), lambda i: (0, i, 0)),
            pl.BlockSpec((1, BM, D), lambda i: (1, i, 0)),
            pl.BlockSpec((BM, 1), lambda i: (i, 0)),
            pl.BlockSpec((D, D), lambda i: (0, 0)),
            pl.BlockSpec((1, D), lambda i: (0, 0)),
            pl.BlockSpec((D, D), lambda i: (0, 0)),
            pl.BlockSpec((1, D), lambda i: (0, 0)),
            pl.BlockSpec((1, D), lambda i: (0, 0)),
            pl.BlockSpec((1, D), lambda i: (0, 0)),
            pl.BlockSpec((1, 1), lambda i: (0, 0)),
        ],
        out_specs=[
            pl.BlockSpec((BM, D), lambda i: (i, 0)),
            pl.BlockSpec((BM, 1), lambda i: (i, 0)),
        ],
        out_shape=[
            jax.ShapeDtypeStruct((N, D), jnp.float32),
            jax.ShapeDtypeStruct((N, 1), jnp.float32),
        ],
    )(x, pa, pa, pb, pb, importance, Wn, bn.reshape(1, D),
      Wg[:D], Wg[D:D + 1], bg.reshape(1, D), Wp.reshape(1, D),
      bp.reshape(1, 1))


_PERM = np.empty((D,), dtype=np.int32)
for _g in range(D // 32):
    for _i in range(16):
        _PERM[32 * _g + 2 * _i] = 32 * _g + _i
        _PERM[32 * _g + 2 * _i + 1] = 32 * _g + 16 + _i


def kernel(x, edge_index, edge_attr, importance, We, be, Wn, bn, Wg, bg, Wp, bp):
    eaT = edge_attr.T
    src, dst = edge_index[0], edge_index[1]
    e_a = _edge_proj(eaT, We, be, 0)
    p_a = _sc_aggregate(x, src, dst, e_a, 0)
    e_b = _edge_proj(eaT, We, be, 1)
    p_b = _sc_aggregate(x, src, dst, e_b, 1)
    out, prop = _tail(x, p_a, p_b, importance, Wn, bn, Wg, bg, Wp, bp)
    return (out, prop)
